# baseline (speedup 1.0000x reference)
import functools

import jax
import jax.numpy as jnp
from jax import lax
from jax.experimental import pallas as pl
from jax.experimental.pallas import tpu as pltpu
from jax.experimental.pallas import tpu_sc as plsc

NC = 2
NS = 16
NW = NC * NS
K = 128
NB = 2


def _make_spmm(NP, EP, F):
    EPW = EP // NW
    nchunks = EPW // K
    nd = nchunks // NB
    rpt = NP // NS
    mesh = plsc.VectorSubcoreMesh(core_axis_name="c", subcore_axis_name="s",
                                  num_cores=NC)

    @functools.partial(
        pl.kernel,
        out_type=jax.ShapeDtypeStruct((NC, NP, F), jnp.float32),
        mesh=mesh,
        scratch_types=[
            pltpu.VMEM((nchunks, K), jnp.int32),
            pltpu.VMEM((nchunks, K), jnp.int32),
            pltpu.VMEM((nchunks, K), jnp.float32),
            [pltpu.VMEM((K, F), jnp.float32)] * NB,
            pltpu.VMEM_SHARED((NP, F), jnp.float32),
            pltpu.SemaphoreType.DMA,
            [pltpu.SemaphoreType.DMA] * NB,
            [pltpu.SemaphoreType.DMA] * NB,
        ],
        compiler_params=pltpu.CompilerParams(use_tc_tiling_on_sc=False),
    )
    def spmm(table, dstg, srcg, evg, out,
             src_v, dst_v, ev_v, rows, accum, sem_i, gsem, ssem):
        c = lax.axis_index("c")
        s = lax.axis_index("s")
        wid = s * NC + c

        pltpu.async_copy(srcg.at[wid], src_v, sem_i)
        pltpu.async_copy(dstg.at[wid], dst_v, sem_i)
        pltpu.async_copy(evg.at[wid], ev_v, sem_i)

        zero = jnp.zeros((16,), jnp.float32)
        for e in range(K):
            for j in range(F // 16):
                rows[0][e, pl.ds(j * 16, 16)] = zero
        for k in range(rpt // K):
            pltpu.sync_copy(rows[0], accum.at[pl.ds(s * rpt + k * K, K)])
        plsc.subcore_barrier()

        pltpu.make_async_copy(srcg.at[wid], src_v, sem_i).wait()
        pltpu.make_async_copy(dstg.at[wid], dst_v, sem_i).wait()
        pltpu.make_async_copy(evg.at[wid], ev_v, sem_i).wait()

        def gather(chunk, r):
            pltpu.async_copy(table.at[src_v.at[chunk]], rows[r], gsem[r])

        def gather_wait(chunk, r):
            pltpu.make_async_copy(table.at[src_v.at[chunk]], rows[r],
                                  gsem[r]).wait()

        def scale(chunk, r):
            for g in range(K // 16):
                evv = ev_v[chunk, pl.ds(g * 16, 16)]
                for t in range(16):
                    e = g * 16 + t
                    evb = jnp.full((16,), evv[t], jnp.float32)
                    for j in range(F // 16):
                        sl = pl.ds(j * 16, 16)
                        rows[r][e, sl] = rows[r][e, sl] * evb

        def scatter(chunk, r):
            pltpu.async_copy(rows[r], accum.at[dst_v.at[chunk]], ssem[r],
                             add=True)

        def scatter_wait(chunk, r):
            pltpu.make_async_copy(rows[r], accum.at[dst_v.at[chunk]],
                                  ssem[r]).wait()

        for r in range(NB):
            gather(r, r)

        def body(i, carry):
            for r in range(NB):
                ck = NB * i + r
                gather_wait(ck, r)
                scale(ck, r)
                scatter(ck, r)
            for r in range(NB):
                @pl.when(i < nd - 1)
                def _():
                    ck = NB * i + r
                    scatter_wait(ck, r)
                    gather(ck + NB, r)
            return carry

        lax.fori_loop(0, nd, body, 0)
        for r in range(NB):
            scatter_wait(NB * (nd - 1) + r, r)
        plsc.subcore_barrier()
        pltpu.sync_copy(accum.at[pl.ds(s * rpt, rpt)],
                        out.at[c, pl.ds(s * rpt, rpt)])

    return spmm


def _mm1(x, w, BM):
    NP, D = x.shape
    H = w.shape[1]

    def body(x_ref, w_ref, o_ref):
        o_ref[...] = jnp.dot(x_ref[...], w_ref[...],
                             preferred_element_type=jnp.float32)

    return pl.pallas_call(
        body,
        grid=(NP // BM,),
        in_specs=[
            pl.BlockSpec((BM, D), lambda i: (i, 0)),
            pl.BlockSpec((D, H), lambda i: (0, 0)),
        ],
        out_specs=pl.BlockSpec((BM, H), lambda i: (i, 0)),
        out_shape=jax.ShapeDtypeStruct((NP, H), jnp.float32),
    )(x, w)


def _mm2_relu(parts, b, w, BM):
    _, NP, H = parts.shape
    CP = w.shape[1]

    def body(p_ref, b_ref, w_ref, o_ref):
        acc = p_ref[0]
        for q in range(1, NC):
            acc = acc + p_ref[q]
        x = jax.nn.relu(acc + b_ref[...])
        o_ref[...] = jnp.dot(x, w_ref[...], preferred_element_type=jnp.float32)

    return pl.pallas_call(
        body,
        grid=(NP // BM,),
        in_specs=[
            pl.BlockSpec((NC, BM, H), lambda i: (0, i, 0)),
            pl.BlockSpec((1, H), lambda i: (0, 0)),
            pl.BlockSpec((H, CP), lambda i: (0, 0)),
        ],
        out_specs=pl.BlockSpec((BM, CP), lambda i: (i, 0)),
        out_shape=jax.ShapeDtypeStruct((NP, CP), jnp.float32),
    )(parts, b, w)


def _combine(parts, b, BM):
    _, NP, CP = parts.shape

    def body(p_ref, b_ref, o_ref):
        acc = p_ref[0]
        for q in range(1, NC):
            acc = acc + p_ref[q]
        o_ref[...] = acc + b_ref[...]

    return pl.pallas_call(
        body,
        grid=(NP // BM,),
        in_specs=[
            pl.BlockSpec((NC, BM, CP), lambda i: (0, i, 0)),
            pl.BlockSpec((1, CP), lambda i: (0, 0)),
        ],
        out_specs=pl.BlockSpec((BM, CP), lambda i: (i, 0)),
        out_shape=jax.ShapeDtypeStruct((NP, CP), jnp.float32),
    )(parts, b)


def kernel(inputs, edge_index, edge_values, W1, b1, W2, b2):
    N, D = inputs.shape
    H = W1.shape[1]
    C = W2.shape[1]
    E = edge_values.shape[0]
    NP = 10240
    CP = 32
    BM = 1280
    EP = -(-E // (NB * K * NW)) * (NB * K * NW)
    nchunks = EP // NW // K

    xp = jnp.pad(inputs, ((0, NP - N), (0, 0)))
    epad = jnp.arange(EP - E, dtype=jnp.int32) % N
    dst = jnp.concatenate([edge_index[0], epad]).reshape(NW, nchunks, K)
    src = jnp.concatenate([edge_index[1], epad]).reshape(NW, nchunks, K)
    ev = jnp.concatenate(
        [edge_values, jnp.zeros((EP - E,), jnp.float32)]).reshape(NW, nchunks, K)
    b1r = b1.reshape(1, H)
    W2p = jnp.pad(W2, ((0, 0), (0, CP - C)))
    b2p = jnp.pad(b2, (0, CP - C)).reshape(1, CP)

    s1 = _mm1(xp, W1, BM)
    p1 = _make_spmm(NP, EP, H)(s1, dst, src, ev)
    s2 = _mm2_relu(p1, b1r, W2p, BM)
    p2 = _make_spmm(NP, EP, CP)(s2, dst, src, ev)
    out = _combine(p2, b2p, BM)
    return out[:N, :C]


# speedup vs baseline: 1.0914x; 1.0914x over previous
import functools

import jax
import jax.numpy as jnp
from jax import lax
from jax.experimental import pallas as pl
from jax.experimental.pallas import tpu as pltpu
from jax.experimental.pallas import tpu_sc as plsc

NC = 2
NS = 16
NW = NC * NS
K = 128
NB = 2


def _make_spmm(NP, E, F):
    nch_total = E // K
    npt = nch_total // NW
    nextra = nch_total - npt * NW
    assert npt % NB == 0 and nch_total % 1 == 0 and E % K == 0
    nd = npt // NB
    rpt = NP // NS
    epw = npt * K
    mesh = plsc.VectorSubcoreMesh(core_axis_name="c", subcore_axis_name="s",
                                  num_cores=NC)

    @functools.partial(
        pl.kernel,
        out_type=jax.ShapeDtypeStruct((NC, NP, F), jnp.float32),
        mesh=mesh,
        scratch_types=[
            pltpu.VMEM(((npt + 1) * K,), jnp.int32),
            pltpu.VMEM(((npt + 1) * K,), jnp.int32),
            pltpu.VMEM(((npt + 1) * K,), jnp.float32),
            [pltpu.VMEM((K, F), jnp.float32)] * NB,
            pltpu.VMEM_SHARED((NP, F), jnp.float32),
            pltpu.SemaphoreType.DMA,
            [pltpu.SemaphoreType.DMA] * NB,
        ],
        compiler_params=pltpu.CompilerParams(use_tc_tiling_on_sc=False),
    )
    def spmm(table, edge, evg, out, src_v, dst_v, ev_v, rows, accum,
             sem_i, gsem):
        c = lax.axis_index("c")
        s = lax.axis_index("s")
        wid = s * NC + c

        base = wid * epw
        pltpu.async_copy(edge.at[1, pl.ds(base, epw)],
                         src_v.at[pl.ds(0, epw)], sem_i)
        pltpu.async_copy(edge.at[0, pl.ds(base, epw)],
                         dst_v.at[pl.ds(0, epw)], sem_i)
        pltpu.async_copy(evg.at[pl.ds(base, epw)],
                         ev_v.at[pl.ds(0, epw)], sem_i)
        xbase = NW * epw + wid * K

        @pl.when(wid < nextra)
        def _():
            pltpu.async_copy(edge.at[1, pl.ds(xbase, K)],
                             src_v.at[pl.ds(epw, K)], sem_i)
            pltpu.async_copy(edge.at[0, pl.ds(xbase, K)],
                             dst_v.at[pl.ds(epw, K)], sem_i)
            pltpu.async_copy(evg.at[pl.ds(xbase, K)],
                             ev_v.at[pl.ds(epw, K)], sem_i)

        zero = jnp.zeros((16,), jnp.float32)
        for e in range(K):
            for j in range(F // 16):
                rows[0][e, pl.ds(j * 16, 16)] = zero
        for k in range(rpt // K):
            pltpu.sync_copy(rows[0], accum.at[pl.ds(s * rpt + k * K, K)])
        plsc.subcore_barrier()

        pltpu.make_async_copy(edge.at[1, pl.ds(base, epw)],
                              src_v.at[pl.ds(0, epw)], sem_i).wait()
        pltpu.make_async_copy(edge.at[0, pl.ds(base, epw)],
                              dst_v.at[pl.ds(0, epw)], sem_i).wait()
        pltpu.make_async_copy(evg.at[pl.ds(base, epw)],
                              ev_v.at[pl.ds(0, epw)], sem_i).wait()

        @pl.when(wid < nextra)
        def _():
            pltpu.make_async_copy(edge.at[1, pl.ds(xbase, K)],
                                  src_v.at[pl.ds(epw, K)], sem_i).wait()
            pltpu.make_async_copy(edge.at[0, pl.ds(xbase, K)],
                                  dst_v.at[pl.ds(epw, K)], sem_i).wait()
            pltpu.make_async_copy(evg.at[pl.ds(xbase, K)],
                                  ev_v.at[pl.ds(epw, K)], sem_i).wait()

        def gather(l, r):
            pltpu.async_copy(table.at[src_v.at[pl.ds(l * K, K)]], rows[r],
                             gsem[r])

        def gather_wait(l, r):
            pltpu.make_async_copy(table.at[src_v.at[pl.ds(l * K, K)]],
                                  rows[r], gsem[r]).wait()

        def scale(l, r):
            for g in range(K // 16):
                evv = ev_v[pl.ds(l * K + g * 16, 16)]
                for t in range(16):
                    e = g * 16 + t
                    evb = jnp.full((16,), evv[t], jnp.float32)
                    for j in range(F // 16):
                        sl = pl.ds(j * 16, 16)
                        rows[r][e, sl] = rows[r][e, sl] * evb

        def scatter(l, r):
            pltpu.sync_copy(rows[r], accum.at[dst_v.at[pl.ds(l * K, K)]],
                            add=True)

        for r in range(NB):
            gather(r, r)

        def body(i, carry):
            for r in range(NB):
                l = NB * i + r
                gather_wait(l, r)
                scale(l, r)
                scatter(l, r)

                @pl.when(i < nd - 1)
                def _():
                    gather(l + NB, r)
            return carry

        lax.fori_loop(0, nd, body, 0)

        @pl.when(wid < nextra)
        def _():
            gather(npt, 0)
            gather_wait(npt, 0)
            scale(npt, 0)
            scatter(npt, 0)

        plsc.subcore_barrier()
        pltpu.sync_copy(accum.at[pl.ds(s * rpt, rpt)],
                        out.at[c, pl.ds(s * rpt, rpt)])

    return spmm


def _mm1(x, w, BM):
    NP, D = x.shape
    H = w.shape[1]

    def body(x_ref, w_ref, o_ref):
        o_ref[...] = jnp.dot(x_ref[...], w_ref[...],
                             preferred_element_type=jnp.float32)

    return pl.pallas_call(
        body,
        grid=(NP // BM,),
        in_specs=[
            pl.BlockSpec((BM, D), lambda i: (i, 0)),
            pl.BlockSpec((D, H), lambda i: (0, 0)),
        ],
        out_specs=pl.BlockSpec((BM, H), lambda i: (i, 0)),
        out_shape=jax.ShapeDtypeStruct((NP, H), jnp.float32),
    )(x, w)


def _mm2_relu(parts, b, w, BM):
    _, NP, H = parts.shape
    CP = w.shape[1]

    def body(p_ref, b_ref, w_ref, o_ref):
        acc = p_ref[0]
        for q in range(1, NC):
            acc = acc + p_ref[q]
        x = jax.nn.relu(acc + b_ref[...])
        o_ref[...] = jnp.dot(x, w_ref[...], preferred_element_type=jnp.float32)

    return pl.pallas_call(
        body,
        grid=(NP // BM,),
        in_specs=[
            pl.BlockSpec((NC, BM, H), lambda i: (0, i, 0)),
            pl.BlockSpec((1, H), lambda i: (0, 0)),
            pl.BlockSpec((H, CP), lambda i: (0, 0)),
        ],
        out_specs=pl.BlockSpec((BM, CP), lambda i: (i, 0)),
        out_shape=jax.ShapeDtypeStruct((NP, CP), jnp.float32),
    )(parts, b, w)


def _combine(parts, b, BM):
    _, NP, CP = parts.shape

    def body(p_ref, b_ref, o_ref):
        acc = p_ref[0]
        for q in range(1, NC):
            acc = acc + p_ref[q]
        o_ref[...] = acc + b_ref[...]

    return pl.pallas_call(
        body,
        grid=(NP // BM,),
        in_specs=[
            pl.BlockSpec((NC, BM, CP), lambda i: (0, i, 0)),
            pl.BlockSpec((1, CP), lambda i: (0, 0)),
        ],
        out_specs=pl.BlockSpec((BM, CP), lambda i: (i, 0)),
        out_shape=jax.ShapeDtypeStruct((NP, CP), jnp.float32),
    )(parts, b)


def kernel(inputs, edge_index, edge_values, W1, b1, W2, b2):
    N, D = inputs.shape
    H = W1.shape[1]
    C = W2.shape[1]
    E = edge_values.shape[0]
    NP = 10240
    CP = 32
    BM = 1280

    xp = jnp.pad(inputs, ((0, NP - N), (0, 0)))
    b1r = b1.reshape(1, H)
    W2p = jnp.pad(W2, ((0, 0), (0, CP - C)))
    b2p = jnp.pad(b2, (0, CP - C)).reshape(1, CP)

    s1 = _mm1(xp, W1, BM)
    p1 = _make_spmm(NP, E, H)(s1, edge_index, edge_values)
    s2 = _mm2_relu(p1, b1r, W2p, BM)
    p2 = _make_spmm(NP, E, CP)(s2, edge_index, edge_values)
    out = _combine(p2, b2p, BM)
    return out[:N, :C]
